# baseline (speedup 1.0000x reference)
import functools

import jax
import jax.numpy as jnp
from jax import lax
from jax.experimental import pallas as pl
from jax.experimental.pallas import tpu as pltpu
from jax.experimental.pallas import tpu_sc as plsc

_N = 10000
_E = 320000
_D = 128
_NC = 2
_NS = 16
_NW = _NC * _NS
_EPT = _E // _NW
_CHUNK = 80
_NCHUNK = _EPT // _CHUNK
_CHUNK_D = 80
_NCHUNK_D = _EPT // _CHUNK_D
_NPAD = 10240
_RPT = _NPAD // _NS

_mesh = plsc.VectorSubcoreMesh(core_axis_name="c", subcore_axis_name="s")


@functools.partial(
    pl.kernel,
    out_type=jax.ShapeDtypeStruct((_NC, _NPAD), jnp.float32),
    mesh=_mesh,
    scratch_types=[
        pltpu.VMEM((_NCHUNK_D, _CHUNK_D), jnp.int32),
        pltpu.VMEM((_CHUNK_D,), jnp.float32),
        pltpu.VMEM((_RPT,), jnp.float32),
        pltpu.VMEM_SHARED((_NPAD,), jnp.float32),
        pltpu.SemaphoreType.DMA,
    ],
)
def _sc_degree(dst_hbm, out_hbm, idx_v, ones_v, stage_v, acc_sh, sem):
    c = lax.axis_index("c")
    s = lax.axis_index("s")
    wid = c * _NS + s

    def _fill_zero(i, carry):
        stage_v[pl.ds(i * 16, 16)] = jnp.zeros((16,), jnp.float32)
        return carry

    lax.fori_loop(0, _RPT // 16, _fill_zero, 0)

    def _fill_one(i, carry):
        ones_v[pl.ds(i * 16, 16)] = jnp.ones((16,), jnp.float32)
        return carry

    lax.fori_loop(0, _CHUNK_D // 16, _fill_one, 0)

    pltpu.sync_copy(dst_hbm.at[wid], idx_v)
    pltpu.sync_copy(stage_v, acc_sh.at[pl.ds(s * _RPT, _RPT)])
    plsc.subcore_barrier()

    def _chunk(k, carry):
        pltpu.sync_copy(ones_v, acc_sh.at[idx_v.at[k]], add=True)
        return carry

    lax.fori_loop(0, _NCHUNK_D, _chunk, 0)
    plsc.subcore_barrier()
    pltpu.sync_copy(acc_sh.at[pl.ds(s * _RPT, _RPT)],
                    out_hbm.at[c, pl.ds(s * _RPT, _RPT)])


_NBUF = 3
_G = 1


@functools.partial(
    pl.kernel,
    out_type=jax.ShapeDtypeStruct((_NC, _NPAD, _D), jnp.float32),
    mesh=_mesh,
    scratch_types=[
        [pltpu.VMEM((_CHUNK,), jnp.int32)] * _NBUF,
        [pltpu.VMEM((_CHUNK,), jnp.int32)] * _NBUF,
        pltpu.VMEM((_NBUF, _CHUNK, _D), jnp.float32),
        pltpu.VMEM_SHARED((_NPAD, _D), jnp.float32),
        [pltpu.SemaphoreType.DMA] * _NBUF,
        [pltpu.SemaphoreType.DMA] * _NBUF,
        [pltpu.SemaphoreType.DMA] * _NBUF,
    ],
)
def _sc_scatter(src_hbm, dst_hbm, g_hbm, out_hbm,
                sidx, didx, rows_v, acc_sh, isems, gsems, ssems):
    c = lax.axis_index("c")
    s = lax.axis_index("s")
    wid = c * _NS + s
    tbase = wid * _NCHUNK

    def _fill_zero(i, carry):
        for j in range(_D // 16):
            rows_v[0, i, pl.ds(j * 16, 16)] = jnp.zeros((16,), jnp.float32)
        return carry

    lax.fori_loop(0, _CHUNK, _fill_zero, 0)

    def _zero_acc(i, carry):
        pltpu.sync_copy(rows_v.at[0],
                        acc_sh.at[pl.ds(s * _RPT + i * _CHUNK, _CHUNK)])
        return carry

    lax.fori_loop(0, _RPT // _CHUNK, _zero_acc, 0)
    plsc.subcore_barrier()

    def _idx_load(k, b, sem):
        pltpu.async_copy(src_hbm.at[tbase + k], sidx[b], sem)
        pltpu.async_copy(dst_hbm.at[tbase + k], didx[b], sem)

    def _idx_wait(b):
        pltpu.make_async_copy(src_hbm.at[0], sidx[b], isems[b]).wait()
        pltpu.make_async_copy(dst_hbm.at[0], didx[b], isems[b]).wait()

    def _gather(k_dyn, b):
        pltpu.async_copy(g_hbm.at[sidx[b]], rows_v.at[b], gsems[b])

    def _gather_wait(b):
        pltpu.make_async_copy(g_hbm.at[sidx[b]], rows_v.at[b],
                              gsems[b]).wait()

    def _scatter_wait(b):
        pltpu.make_async_copy(rows_v.at[b], acc_sh.at[didx[b]],
                              ssems[b]).wait()

    for k0 in range(_G + 1):
        _idx_load(k0, k0, isems[k0])
    for k0 in range(_G):
        _idx_wait(k0)
        _gather(k0, k0)

    def _group(kg, carry):
        for b in range(_NBUF):
            t = kg * _NBUF + b
            _gather_wait(b)
            pltpu.async_copy(rows_v.at[b], acc_sh.at[didx[b]],
                             ssems[b], add=True)
            br = (b + _G + 1) % _NBUF

            @pl.when(t >= _NBUF - _G - 1)
            def _():
                _scatter_wait(br)

            @pl.when(t + _G + 1 <= _NCHUNK - 1)
            def _():
                _idx_load(t + _G + 1, br, isems[br])

            bg = (b + _G) % _NBUF

            @pl.when(t + _G <= _NCHUNK - 1)
            def _():
                _idx_wait(bg)
                _gather(t + _G, bg)
        return carry

    lax.fori_loop(0, _NCHUNK // _NBUF, _group, 0)
    for t0 in range((_NCHUNK // _NBUF) * _NBUF, _NCHUNK):
        b0 = t0 % _NBUF
        _gather_wait(b0)
        pltpu.async_copy(rows_v.at[b0], acc_sh.at[didx[b0]],
                         ssems[b0], add=True)
        br0 = (b0 + _G + 1) % _NBUF
        if t0 >= _NBUF - _G - 1:
            _scatter_wait(br0)
        if t0 + _G + 1 <= _NCHUNK - 1:
            _idx_load(t0 + _G + 1, br0, isems[br0])
        bg0 = (b0 + _G) % _NBUF
        if t0 + _G <= _NCHUNK - 1:
            _idx_wait(bg0)
            _gather(t0 + _G, bg0)
    for k0 in range(_NCHUNK - (_NBUF - _G - 1), _NCHUNK):
        _scatter_wait(k0 % _NBUF)
    plsc.subcore_barrier()

    def _writeout(i, carry):
        pltpu.sync_copy(acc_sh.at[pl.ds(s * _RPT + i * 128, 128)],
                        out_hbm.at[c, pl.ds(s * _RPT + i * 128, 128)])
        return carry

    lax.fori_loop(0, _RPT // 128, _writeout, 0)


_BM = 1000


def _tc_scale_body(x_ref, w_ref, degp_ref, g_ref, dis_ref):
    h = jnp.dot(x_ref[...], w_ref[...], preferred_element_type=jnp.float32)
    deg = 1.0 + degp_ref[0] + degp_ref[1]
    dis = lax.rsqrt(deg)
    dis_ref[...] = dis
    g_ref[...] = h * dis


_tc_scale = pl.pallas_call(
    _tc_scale_body,
    grid=(_N // _BM,),
    in_specs=[
        pl.BlockSpec((_BM, _D), lambda i: (i, 0)),
        pl.BlockSpec((_D, _D), lambda i: (0, 0)),
        pl.BlockSpec((_NC, _BM, 1), lambda i: (0, i, 0)),
    ],
    out_specs=[
        pl.BlockSpec((_BM, _D), lambda i: (i, 0)),
        pl.BlockSpec((_BM, 1), lambda i: (i, 0)),
    ],
    out_shape=[
        jax.ShapeDtypeStruct((_N, _D), jnp.float32),
        jax.ShapeDtypeStruct((_N, 1), jnp.float32),
    ],
)


def _tc_final_body(p_ref, g_ref, dis_ref, b_ref, o_ref):
    acc = p_ref[0] + p_ref[1] + g_ref[...]
    o_ref[...] = jnp.maximum(acc * dis_ref[...] + b_ref[...], 0.0)


_tc_final = pl.pallas_call(
    _tc_final_body,
    grid=(_N // _BM,),
    in_specs=[
        pl.BlockSpec((_NC, _BM, _D), lambda i: (0, i, 0)),
        pl.BlockSpec((_BM, _D), lambda i: (i, 0)),
        pl.BlockSpec((_BM, 1), lambda i: (i, 0)),
        pl.BlockSpec((1, _D), lambda i: (0, 0)),
    ],
    out_specs=pl.BlockSpec((_BM, _D), lambda i: (i, 0)),
    out_shape=jax.ShapeDtypeStruct((_N, _D), jnp.float32),
)


def kernel(x, edge_index, W, b):
    src2 = edge_index[0].reshape(_NW * _NCHUNK, _CHUNK)
    dst2 = edge_index[1].reshape(_NW * _NCHUNK, _CHUNK)
    dst3 = edge_index[1].reshape(_NW, _NCHUNK_D, _CHUNK_D)
    deg_p = _sc_degree(dst3)
    degp3 = deg_p[:, :_N, None]
    g, dis = _tc_scale(x, W, degp3)
    p = _sc_scatter(src2, dst2, g)
    out = _tc_final(p, g, dis, b.reshape(1, _D))
    return out


# speedup vs baseline: 1.2887x; 1.2887x over previous
import functools

import jax
import jax.numpy as jnp
from jax import lax
from jax.experimental import pallas as pl
from jax.experimental.pallas import tpu as pltpu
from jax.experimental.pallas import tpu_sc as plsc

_N = 10000
_E = 320000
_D = 128
_NC = 2
_NS = 16
_NW = _NC * _NS
_EPT = _E // _NW
_CHUNK = 40
_NCHUNK = _EPT // _CHUNK
_CHUNK_D = 80
_NCHUNK_D = _EPT // _CHUNK_D
_NPAD = 10240
_RPT = _NPAD // _NS

_mesh = plsc.VectorSubcoreMesh(core_axis_name="c", subcore_axis_name="s")


@functools.partial(
    pl.kernel,
    out_type=jax.ShapeDtypeStruct((_NC, _NPAD), jnp.float32),
    mesh=_mesh,
    scratch_types=[
        pltpu.VMEM((_NCHUNK_D, _CHUNK_D), jnp.int32),
        pltpu.VMEM((_CHUNK_D,), jnp.float32),
        pltpu.VMEM((_RPT,), jnp.float32),
        pltpu.VMEM_SHARED((_NPAD,), jnp.float32),
        pltpu.SemaphoreType.DMA,
    ],
)
def _sc_degree(dst_hbm, out_hbm, idx_v, ones_v, stage_v, acc_sh, sem):
    c = lax.axis_index("c")
    s = lax.axis_index("s")
    wid = c * _NS + s

    def _fill_zero(i, carry):
        stage_v[pl.ds(i * 16, 16)] = jnp.zeros((16,), jnp.float32)
        return carry

    lax.fori_loop(0, _RPT // 16, _fill_zero, 0)

    def _fill_one(i, carry):
        ones_v[pl.ds(i * 16, 16)] = jnp.ones((16,), jnp.float32)
        return carry

    lax.fori_loop(0, _CHUNK_D // 16, _fill_one, 0)

    pltpu.sync_copy(dst_hbm.at[wid], idx_v)
    pltpu.sync_copy(stage_v, acc_sh.at[pl.ds(s * _RPT, _RPT)])
    plsc.subcore_barrier()

    def _chunk(k, carry):
        pltpu.sync_copy(ones_v, acc_sh.at[idx_v.at[k]], add=True)
        return carry

    lax.fori_loop(0, _NCHUNK_D, _chunk, 0)
    plsc.subcore_barrier()
    pltpu.sync_copy(acc_sh.at[pl.ds(s * _RPT, _RPT)],
                    out_hbm.at[c, pl.ds(s * _RPT, _RPT)])


_NBUF = 6
_G = 4


@functools.partial(
    pl.kernel,
    out_type=jax.ShapeDtypeStruct((_NC, _NPAD, _D), jnp.float32),
    mesh=_mesh,
    scratch_types=[
        [pltpu.VMEM((_CHUNK,), jnp.int32)] * _NBUF,
        [pltpu.VMEM((_CHUNK,), jnp.int32)] * _NBUF,
        pltpu.VMEM((_NBUF, _CHUNK, _D), jnp.float32),
        pltpu.VMEM_SHARED((_NPAD, _D), jnp.float32),
        [pltpu.SemaphoreType.DMA] * _NBUF,
        [pltpu.SemaphoreType.DMA] * _NBUF,
        [pltpu.SemaphoreType.DMA] * _NBUF,
    ],
)
def _sc_scatter(src_hbm, dst_hbm, g_hbm, out_hbm,
                sidx, didx, rows_v, acc_sh, isems, gsems, ssems):
    c = lax.axis_index("c")
    s = lax.axis_index("s")
    wid = c * _NS + s
    tbase = wid * _NCHUNK

    def _fill_zero(i, carry):
        for j in range(_D // 16):
            rows_v[0, i, pl.ds(j * 16, 16)] = jnp.zeros((16,), jnp.float32)
        return carry

    lax.fori_loop(0, _CHUNK, _fill_zero, 0)

    def _zero_acc(i, carry):
        pltpu.sync_copy(rows_v.at[0],
                        acc_sh.at[pl.ds(s * _RPT + i * _CHUNK, _CHUNK)])
        return carry

    lax.fori_loop(0, _RPT // _CHUNK, _zero_acc, 0)
    plsc.subcore_barrier()

    def _idx_load(k, b, sem):
        pltpu.async_copy(src_hbm.at[tbase + k], sidx[b], sem)
        pltpu.async_copy(dst_hbm.at[tbase + k], didx[b], sem)

    def _idx_wait(b):
        pltpu.make_async_copy(src_hbm.at[0], sidx[b], isems[b]).wait()
        pltpu.make_async_copy(dst_hbm.at[0], didx[b], isems[b]).wait()

    def _gather(k_dyn, b):
        pltpu.async_copy(g_hbm.at[sidx[b]], rows_v.at[b], gsems[b])

    def _gather_wait(b):
        pltpu.make_async_copy(g_hbm.at[sidx[b]], rows_v.at[b],
                              gsems[b]).wait()

    def _scatter_wait(b):
        pltpu.make_async_copy(rows_v.at[b], acc_sh.at[didx[b]],
                              ssems[b]).wait()

    for k0 in range(_G + 1):
        _idx_load(k0, k0, isems[k0])
    for k0 in range(_G):
        _idx_wait(k0)
        _gather(k0, k0)

    def _group(kg, carry):
        for b in range(_NBUF):
            t = kg * _NBUF + b
            _gather_wait(b)
            pltpu.async_copy(rows_v.at[b], acc_sh.at[didx[b]],
                             ssems[b], add=True)
            br = (b + _G + 1) % _NBUF

            @pl.when(t >= _NBUF - _G - 1)
            def _():
                _scatter_wait(br)

            @pl.when(t + _G + 1 <= _NCHUNK - 1)
            def _():
                _idx_load(t + _G + 1, br, isems[br])

            bg = (b + _G) % _NBUF

            @pl.when(t + _G <= _NCHUNK - 1)
            def _():
                _idx_wait(bg)
                _gather(t + _G, bg)
        return carry

    lax.fori_loop(0, _NCHUNK // _NBUF, _group, 0)
    for t0 in range((_NCHUNK // _NBUF) * _NBUF, _NCHUNK):
        b0 = t0 % _NBUF
        _gather_wait(b0)
        pltpu.async_copy(rows_v.at[b0], acc_sh.at[didx[b0]],
                         ssems[b0], add=True)
        br0 = (b0 + _G + 1) % _NBUF
        if t0 >= _NBUF - _G - 1:
            _scatter_wait(br0)
        if t0 + _G + 1 <= _NCHUNK - 1:
            _idx_load(t0 + _G + 1, br0, isems[br0])
        bg0 = (b0 + _G) % _NBUF
        if t0 + _G <= _NCHUNK - 1:
            _idx_wait(bg0)
            _gather(t0 + _G, bg0)
    for k0 in range(_NCHUNK - (_NBUF - _G - 1), _NCHUNK):
        _scatter_wait(k0 % _NBUF)
    plsc.subcore_barrier()

    def _writeout(i, carry):
        pltpu.sync_copy(acc_sh.at[pl.ds(s * _RPT + i * 128, 128)],
                        out_hbm.at[c, pl.ds(s * _RPT + i * 128, 128)])
        return carry

    lax.fori_loop(0, _RPT // 128, _writeout, 0)


_BM = 1000


def _tc_scale_body(x_ref, w_ref, degp_ref, g_ref, dis_ref):
    h = jnp.dot(x_ref[...], w_ref[...], preferred_element_type=jnp.float32)
    deg = 1.0 + degp_ref[0] + degp_ref[1]
    dis = lax.rsqrt(deg)
    dis_ref[...] = dis
    g_ref[...] = h * dis


_tc_scale = pl.pallas_call(
    _tc_scale_body,
    grid=(_N // _BM,),
    in_specs=[
        pl.BlockSpec((_BM, _D), lambda i: (i, 0)),
        pl.BlockSpec((_D, _D), lambda i: (0, 0)),
        pl.BlockSpec((_NC, _BM, 1), lambda i: (0, i, 0)),
    ],
    out_specs=[
        pl.BlockSpec((_BM, _D), lambda i: (i, 0)),
        pl.BlockSpec((_BM, 1), lambda i: (i, 0)),
    ],
    out_shape=[
        jax.ShapeDtypeStruct((_N, _D), jnp.float32),
        jax.ShapeDtypeStruct((_N, 1), jnp.float32),
    ],
)


def _tc_final_body(p_ref, g_ref, dis_ref, b_ref, o_ref):
    acc = p_ref[0] + p_ref[1] + g_ref[...]
    o_ref[...] = jnp.maximum(acc * dis_ref[...] + b_ref[...], 0.0)


_tc_final = pl.pallas_call(
    _tc_final_body,
    grid=(_N // _BM,),
    in_specs=[
        pl.BlockSpec((_NC, _BM, _D), lambda i: (0, i, 0)),
        pl.BlockSpec((_BM, _D), lambda i: (i, 0)),
        pl.BlockSpec((_BM, 1), lambda i: (i, 0)),
        pl.BlockSpec((1, _D), lambda i: (0, 0)),
    ],
    out_specs=pl.BlockSpec((_BM, _D), lambda i: (i, 0)),
    out_shape=jax.ShapeDtypeStruct((_N, _D), jnp.float32),
)


def kernel(x, edge_index, W, b):
    src2 = edge_index[0].reshape(_NW * _NCHUNK, _CHUNK)
    dst2 = edge_index[1].reshape(_NW * _NCHUNK, _CHUNK)
    dst3 = edge_index[1].reshape(_NW, _NCHUNK_D, _CHUNK_D)
    deg_p = _sc_degree(dst3)
    degp3 = deg_p[:, :_N, None]
    g, dis = _tc_scale(x, W, degp3)
    p = _sc_scatter(src2, dst2, g)
    out = _tc_final(p, g, dis, b.reshape(1, _D))
    return out


# speedup vs baseline: 1.3114x; 1.0176x over previous
import functools

import jax
import jax.numpy as jnp
from jax import lax
from jax.experimental import pallas as pl
from jax.experimental.pallas import tpu as pltpu
from jax.experimental.pallas import tpu_sc as plsc

_N = 10000
_E = 320000
_D = 128
_NC = 2
_NS = 16
_NW = _NC * _NS
_EPT = _E // _NW
_CHUNK = 40
_NCHUNK = _EPT // _CHUNK
_CHUNK_D = 80
_NCHUNK_D = _EPT // _CHUNK_D
_NPAD = 10240
_RPT = _NPAD // _NS

_mesh = plsc.VectorSubcoreMesh(core_axis_name="c", subcore_axis_name="s")


@functools.partial(
    pl.kernel,
    out_type=jax.ShapeDtypeStruct((_NC, _NPAD), jnp.float32),
    mesh=_mesh,
    scratch_types=[
        pltpu.VMEM((_NCHUNK_D, _CHUNK_D), jnp.int32),
        pltpu.VMEM((_CHUNK_D,), jnp.float32),
        pltpu.VMEM((_RPT,), jnp.float32),
        pltpu.VMEM_SHARED((_NPAD,), jnp.float32),
        pltpu.SemaphoreType.DMA,
    ],
)
def _sc_degree(dst_hbm, out_hbm, idx_v, ones_v, stage_v, acc_sh, sem):
    c = lax.axis_index("c")
    s = lax.axis_index("s")
    wid = c * _NS + s

    def _fill_zero(i, carry):
        stage_v[pl.ds(i * 16, 16)] = jnp.zeros((16,), jnp.float32)
        return carry

    lax.fori_loop(0, _RPT // 16, _fill_zero, 0)

    def _fill_one(i, carry):
        ones_v[pl.ds(i * 16, 16)] = jnp.ones((16,), jnp.float32)
        return carry

    lax.fori_loop(0, _CHUNK_D // 16, _fill_one, 0)

    pltpu.sync_copy(dst_hbm.at[wid], idx_v)
    pltpu.sync_copy(stage_v, acc_sh.at[pl.ds(s * _RPT, _RPT)])
    plsc.subcore_barrier()

    def _chunk(k, carry):
        pltpu.sync_copy(ones_v, acc_sh.at[idx_v.at[k]], add=True)
        return carry

    lax.fori_loop(0, _NCHUNK_D, _chunk, 0)
    plsc.subcore_barrier()
    pltpu.sync_copy(acc_sh.at[pl.ds(s * _RPT, _RPT)],
                    out_hbm.at[c, pl.ds(s * _RPT, _RPT)])


_NBUF = 7
_G = 5


@functools.partial(
    pl.kernel,
    out_type=jax.ShapeDtypeStruct((_NC, _NPAD, _D), jnp.float32),
    mesh=_mesh,
    scratch_types=[
        [pltpu.VMEM((_CHUNK,), jnp.int32)] * _NBUF,
        [pltpu.VMEM((_CHUNK,), jnp.int32)] * _NBUF,
        pltpu.VMEM((_NBUF, _CHUNK, _D), jnp.float32),
        pltpu.VMEM_SHARED((_NPAD, _D), jnp.float32),
        [pltpu.SemaphoreType.DMA] * _NBUF,
        [pltpu.SemaphoreType.DMA] * _NBUF,
        [pltpu.SemaphoreType.DMA] * _NBUF,
    ],
)
def _sc_scatter(src_hbm, dst_hbm, g_hbm, out_hbm,
                sidx, didx, rows_v, acc_sh, isems, gsems, ssems):
    c = lax.axis_index("c")
    s = lax.axis_index("s")
    wid = c * _NS + s
    tbase = wid * _NCHUNK

    def _fill_zero(i, carry):
        for j in range(_D // 16):
            rows_v[0, i, pl.ds(j * 16, 16)] = jnp.zeros((16,), jnp.float32)
        return carry

    lax.fori_loop(0, _CHUNK, _fill_zero, 0)

    def _zero_acc(i, carry):
        pltpu.sync_copy(rows_v.at[0],
                        acc_sh.at[pl.ds(s * _RPT + i * _CHUNK, _CHUNK)])
        return carry

    lax.fori_loop(0, _RPT // _CHUNK, _zero_acc, 0)
    plsc.subcore_barrier()

    def _idx_load(k, b, sem):
        pltpu.async_copy(src_hbm.at[tbase + k], sidx[b], sem)
        pltpu.async_copy(dst_hbm.at[tbase + k], didx[b], sem)

    def _idx_wait(b):
        pltpu.make_async_copy(src_hbm.at[0], sidx[b], isems[b]).wait()
        pltpu.make_async_copy(dst_hbm.at[0], didx[b], isems[b]).wait()

    def _gather(k_dyn, b):
        pltpu.async_copy(g_hbm.at[sidx[b]], rows_v.at[b], gsems[b])

    def _gather_wait(b):
        pltpu.make_async_copy(g_hbm.at[sidx[b]], rows_v.at[b],
                              gsems[b]).wait()

    def _scatter_wait(b):
        pltpu.make_async_copy(rows_v.at[b], acc_sh.at[didx[b]],
                              ssems[b]).wait()

    for k0 in range(_G + 1):
        _idx_load(k0, k0, isems[k0])
    for k0 in range(_G):
        _idx_wait(k0)
        _gather(k0, k0)

    def _group(kg, carry):
        for b in range(_NBUF):
            t = kg * _NBUF + b
            _gather_wait(b)
            pltpu.async_copy(rows_v.at[b], acc_sh.at[didx[b]],
                             ssems[b], add=True)
            br = (b + _G + 1) % _NBUF

            @pl.when(t >= _NBUF - _G - 1)
            def _():
                _scatter_wait(br)

            @pl.when(t + _G + 1 <= _NCHUNK - 1)
            def _():
                _idx_load(t + _G + 1, br, isems[br])

            bg = (b + _G) % _NBUF

            @pl.when(t + _G <= _NCHUNK - 1)
            def _():
                _idx_wait(bg)
                _gather(t + _G, bg)
        return carry

    lax.fori_loop(0, _NCHUNK // _NBUF, _group, 0)
    for t0 in range((_NCHUNK // _NBUF) * _NBUF, _NCHUNK):
        b0 = t0 % _NBUF
        _gather_wait(b0)
        pltpu.async_copy(rows_v.at[b0], acc_sh.at[didx[b0]],
                         ssems[b0], add=True)
        br0 = (b0 + _G + 1) % _NBUF
        if t0 >= _NBUF - _G - 1:
            _scatter_wait(br0)
        if t0 + _G + 1 <= _NCHUNK - 1:
            _idx_load(t0 + _G + 1, br0, isems[br0])
        bg0 = (b0 + _G) % _NBUF
        if t0 + _G <= _NCHUNK - 1:
            _idx_wait(bg0)
            _gather(t0 + _G, bg0)
    for k0 in range(_NCHUNK - (_NBUF - _G - 1), _NCHUNK):
        _scatter_wait(k0 % _NBUF)
    plsc.subcore_barrier()

    def _writeout(i, carry):
        pltpu.sync_copy(acc_sh.at[pl.ds(s * _RPT + i * 128, 128)],
                        out_hbm.at[c, pl.ds(s * _RPT + i * 128, 128)])
        return carry

    lax.fori_loop(0, _RPT // 128, _writeout, 0)


_BM = 1000


def _tc_scale_body(x_ref, w_ref, degp_ref, g_ref, dis_ref):
    h = jnp.dot(x_ref[...], w_ref[...], preferred_element_type=jnp.float32)
    deg = 1.0 + degp_ref[0] + degp_ref[1]
    dis = lax.rsqrt(deg)
    dis_ref[...] = dis
    g_ref[...] = h * dis


_tc_scale = pl.pallas_call(
    _tc_scale_body,
    grid=(_N // _BM,),
    in_specs=[
        pl.BlockSpec((_BM, _D), lambda i: (i, 0)),
        pl.BlockSpec((_D, _D), lambda i: (0, 0)),
        pl.BlockSpec((_NC, _BM, 1), lambda i: (0, i, 0)),
    ],
    out_specs=[
        pl.BlockSpec((_BM, _D), lambda i: (i, 0)),
        pl.BlockSpec((_BM, 1), lambda i: (i, 0)),
    ],
    out_shape=[
        jax.ShapeDtypeStruct((_N, _D), jnp.float32),
        jax.ShapeDtypeStruct((_N, 1), jnp.float32),
    ],
)


def _tc_final_body(p_ref, g_ref, dis_ref, b_ref, o_ref):
    acc = p_ref[0] + p_ref[1] + g_ref[...]
    o_ref[...] = jnp.maximum(acc * dis_ref[...] + b_ref[...], 0.0)


_tc_final = pl.pallas_call(
    _tc_final_body,
    grid=(_N // _BM,),
    in_specs=[
        pl.BlockSpec((_NC, _BM, _D), lambda i: (0, i, 0)),
        pl.BlockSpec((_BM, _D), lambda i: (i, 0)),
        pl.BlockSpec((_BM, 1), lambda i: (i, 0)),
        pl.BlockSpec((1, _D), lambda i: (0, 0)),
    ],
    out_specs=pl.BlockSpec((_BM, _D), lambda i: (i, 0)),
    out_shape=jax.ShapeDtypeStruct((_N, _D), jnp.float32),
)


def kernel(x, edge_index, W, b):
    src2 = edge_index[0].reshape(_NW * _NCHUNK, _CHUNK)
    dst2 = edge_index[1].reshape(_NW * _NCHUNK, _CHUNK)
    dst3 = edge_index[1].reshape(_NW, _NCHUNK_D, _CHUNK_D)
    deg_p = _sc_degree(dst3)
    degp3 = deg_p[:, :_N, None]
    g, dis = _tc_scale(x, W, degp3)
    p = _sc_scatter(src2, dst2, g)
    out = _tc_final(p, g, dis, b.reshape(1, _D))
    return out
